# baseline (speedup 1.0000x reference)
import functools

import jax
import jax.numpy as jnp
from jax import lax
from jax.experimental import pallas as pl
from jax.experimental.pallas import tpu as pltpu
from jax.experimental.pallas import tpu_sc as plsc

N = 50000
E = 800000
G = 256
HID = 64
NPAD = 50176
NC = 2
NS = 16
BN_S = 1.0 / (1.0 + 1e-5) ** 0.5

f32 = jnp.float32
i32 = jnp.int32

_SC_MESH = plsc.VectorSubcoreMesh(core_axis_name="c", subcore_axis_name="s")
_SC_PARAMS = pltpu.CompilerParams(use_tc_tiling_on_sc=False)

EC = 80
EROWS = E // EC
T_EDGES = E // NS
TROWS = T_EDGES // EC
GRP = 5
T_GRPS = TROWS // GRP
NPAIR = (T_GRPS - 1) // 2
DGRP = 25

RPT = NPAD // NS
ZB = 56
ZCOP = RPT // ZB


def _zero2d(ref, nrows, ncols):
  def body(r, _):
    for cc in range(ncols // 16):
      ref[r, pl.ds(cc * 16, 16)] = jnp.zeros((16,), f32)
    return 0
  lax.fori_loop(0, nrows, body, 0)


def _deg_body(e3d, out, ones_v, didx, zbuf, dacc, sem):
  c = lax.axis_index("c")
  t = lax.axis_index("s")
  for j in range(EC // 16):
    ones_v[pl.ds(j * 16, 16)] = jnp.ones((16,), f32)
  def zb(i, _):
    zbuf[pl.ds(i * 16, 16)] = jnp.zeros((16,), f32)
    return 0
  lax.fori_loop(0, RPT // 16, zb, 0)
  pltpu.sync_copy(zbuf, dacc.at[pl.ds(t * RPT, RPT)])
  plsc.subcore_barrier()
  def grp(i, _):
    rbase = t * TROWS + i * DGRP
    pltpu.sync_copy(e3d.at[c, pl.ds(rbase, DGRP)], didx)
    descs = [
        pltpu.async_copy(ones_v, dacc.at[didx.at[j]], sem, add=True)
        for j in range(DGRP)
    ]
    for d in descs:
      d.wait()
    return 0
  lax.fori_loop(0, TROWS // DGRP, grp, 0)
  plsc.subcore_barrier()
  pltpu.sync_copy(dacc.at[pl.ds(t * RPT, RPT)], out.at[c, pl.ds(t * RPT, RPT)])


_deg_call = pl.kernel(
    _deg_body,
    out_type=jax.ShapeDtypeStruct((NC, NPAD), f32),
    mesh=_SC_MESH,
    compiler_params=_SC_PARAMS,
    scratch_types=[
        pltpu.VMEM((EC,), f32),
        pltpu.VMEM((DGRP, EC), i32),
        pltpu.VMEM((RPT,), f32),
        pltpu.VMEM_SHARED((NPAD,), f32),
        pltpu.SemaphoreType.DMA,
    ],
)


def _scat_body(xs, src3d, dst3d, out,
               sidxA, didxA, rowsA, sidxB, didxB, rowsB, zbuf, acc,
               gsemA, ssemA, isemA, gsemB, ssemB, isemB):
  c = lax.axis_index("c")
  t = lax.axis_index("s")
  _zero2d(zbuf, ZB, 32)
  zd = [pltpu.async_copy(zbuf, acc.at[pl.ds((t * ZCOP + k) * ZB, ZB)], ssemA)
        for k in range(ZCOP)]
  for d in zd:
    d.wait()
  plsc.subcore_barrier()

  def fire_idx(g, sidx, didx, isem):
    rbase = t * TROWS + g * GRP
    return (pltpu.async_copy(src3d.at[c, pl.ds(rbase, GRP)], sidx, isem),
            pltpu.async_copy(dst3d.at[pl.ds(rbase, GRP)], didx, isem))

  def fire_gathers(sidx, rows, gsem):
    for j in range(GRP):
      pltpu.async_copy(xs.at[sidx.at[j]], rows.at[pl.ds(j * EC, EC)], gsem)

  def drain_gathers(sidx, rows, gsem):
    for j in range(GRP):
      pltpu.make_async_copy(xs.at[sidx.at[j]], rows.at[pl.ds(j * EC, EC)],
                            gsem).wait()

  def fire_scats(rows, didx, ssem):
    for j in range(GRP):
      pltpu.async_copy(rows.at[pl.ds(j * EC, EC)], acc.at[didx.at[j]], ssem,
                       add=True)

  def drain_scats(rows, ssem):
    for j in range(GRP):
      pltpu.make_async_copy(xs.at[pl.ds(0, EC)], rows.at[pl.ds(j * EC, EC)],
                            ssem).wait()

  i0, i1 = fire_idx(0, sidxA, didxA, isemA)
  i0.wait()
  i1.wait()
  fire_gathers(sidxA, rowsA, gsemA)

  def pair(i, _):
    gB = 2 * i + 1
    ib0, ib1 = fire_idx(gB, sidxB, didxB, isemB)
    drain_gathers(sidxA, rowsA, gsemA)
    fire_scats(rowsA, didxA, ssemA)
    ib0.wait()
    ib1.wait()
    fire_gathers(sidxB, rowsB, gsemB)
    drain_scats(rowsA, ssemA)
    ia0, ia1 = fire_idx(gB + 1, sidxA, didxA, isemA)
    drain_gathers(sidxB, rowsB, gsemB)
    fire_scats(rowsB, didxB, ssemB)
    ia0.wait()
    ia1.wait()
    fire_gathers(sidxA, rowsA, gsemA)
    drain_scats(rowsB, ssemB)
    return 0
  lax.fori_loop(0, NPAIR, pair, 0)

  drain_gathers(sidxA, rowsA, gsemA)
  fire_scats(rowsA, didxA, ssemA)
  drain_scats(rowsA, ssemA)

  plsc.subcore_barrier()
  wb = [pltpu.async_copy(acc.at[pl.ds((t * ZCOP + k) * ZB, ZB)],
                         out.at[pl.ds((t * ZCOP + k) * ZB, ZB), c], gsemA)
        for k in range(ZCOP)]
  for d in wb:
    d.wait()


_scat_call = pl.kernel(
    _scat_body,
    out_type=jax.ShapeDtypeStruct((NPAD, NC, 32), f32),
    mesh=_SC_MESH,
    compiler_params=_SC_PARAMS,
    scratch_types=[
        pltpu.VMEM((GRP, EC), i32),
        pltpu.VMEM((GRP, EC), i32),
        pltpu.VMEM((GRP * EC, 32), f32),
        pltpu.VMEM((GRP, EC), i32),
        pltpu.VMEM((GRP, EC), i32),
        pltpu.VMEM((GRP * EC, 32), f32),
        pltpu.VMEM((ZB, 32), f32),
        pltpu.VMEM_SHARED((NPAD, 32), f32),
        pltpu.SemaphoreType.DMA,
        pltpu.SemaphoreType.DMA,
        pltpu.SemaphoreType.DMA,
        pltpu.SemaphoreType.DMA,
        pltpu.SemaphoreType.DMA,
        pltpu.SemaphoreType.DMA,
    ],
)


SEGC = 112
SEG_TROWS = NPAD // (NC * NS) // SEGC

def _seg_body(z, gidflat, out, didx, rows, zbuf, acc):
  c = lax.axis_index("c")
  t = lax.axis_index("s")
  w = c * NS + t
  _zero2d(zbuf, 16, HID)
  pltpu.sync_copy(zbuf, acc.at[pl.ds(t * 16, 16)])
  plsc.subcore_barrier()
  def body(i, _):
    start = (w * SEG_TROWS + i) * SEGC
    pltpu.sync_copy(gidflat.at[pl.ds(start, SEGC)], didx)
    pltpu.sync_copy(z.at[pl.ds(start, SEGC)], rows)
    pltpu.sync_copy(rows, acc.at[didx], add=True)
    return 0
  lax.fori_loop(0, SEG_TROWS, body, 0)
  plsc.subcore_barrier()
  pltpu.sync_copy(acc.at[pl.ds(t * 16, 16)], out.at[c, pl.ds(t * 16, 16)])


_seg_call = pl.kernel(
    _seg_body,
    out_type=jax.ShapeDtypeStruct((NC, G, HID), f32),
    mesh=_SC_MESH,
    compiler_params=_SC_PARAMS,
    scratch_types=[
        pltpu.VMEM((SEGC,), i32),
        pltpu.VMEM((SEGC, HID), f32),
        pltpu.VMEM((16, HID), f32),
        pltpu.VMEM_SHARED((G, HID), f32),
    ],
)


EB = 80000

def _shift_body(e_ref, o_ref):
  s = e_ref[0]
  o_ref[0:1] = 2 * s
  o_ref[1:2] = 2 * s + 1


def _shift_call(edge_index):
  return pl.pallas_call(
      _shift_body,
      grid=(E // EB,),
      in_specs=[pl.BlockSpec((1, 1, EB), lambda i: (0, 0, i))],
      out_specs=pl.BlockSpec((2, EB), lambda i: (0, i)),
      out_shape=jax.ShapeDtypeStruct((2, E), i32),
  )(edge_index.reshape(2, 1, E))


R = 512
NB = NPAD // R


def _l1_body(h_ref, d_ref, w_ref, o_ref):
  dois = lax.rsqrt(jnp.maximum(d_ref[...], 1.0))
  o_ref[...] = jnp.dot(h_ref[...], w_ref[...], preferred_element_type=f32) * dois


def _l1_call(h_pad, degO, W1):
  return pl.pallas_call(
      _l1_body,
      grid=(NB,),
      in_specs=[
          pl.BlockSpec((R, 38), lambda i: (i, 0)),
          pl.BlockSpec((R, 1), lambda i: (i, 0)),
          pl.BlockSpec((38, HID), lambda i: (0, 0)),
      ],
      out_specs=pl.BlockSpec((R, HID), lambda i: (i, 0)),
      out_shape=jax.ShapeDtypeStruct((NPAD, HID), f32),
  )(h_pad, degO, W1)


def _post(a_ref, dI_ref, p_ref):
  diis = lax.rsqrt(jnp.maximum(dI_ref[...], 1.0))
  t = a_ref[...] * diis + p_ref[0:1, :]
  t = t * (BN_S * p_ref[1:2, :]) + p_ref[2:3, :]
  return jnp.maximum(t, 0.0)


def _mid_body(a_ref, dI_ref, dO_ref, w_ref, p_ref, o_ref):
  t = _post(a_ref, dI_ref, p_ref)
  dois = lax.rsqrt(jnp.maximum(dO_ref[...], 1.0))
  o_ref[...] = jnp.dot(t, w_ref[...], preferred_element_type=f32) * dois


def _mid_call(agg, degI, degO, W, p):
  return pl.pallas_call(
      _mid_body,
      grid=(NB,),
      in_specs=[
          pl.BlockSpec((R, HID), lambda i: (i, 0)),
          pl.BlockSpec((R, 1), lambda i: (i, 0)),
          pl.BlockSpec((R, 1), lambda i: (i, 0)),
          pl.BlockSpec((HID, HID), lambda i: (0, 0)),
          pl.BlockSpec((3, HID), lambda i: (0, 0)),
      ],
      out_specs=pl.BlockSpec((R, HID), lambda i: (i, 0)),
      out_shape=jax.ShapeDtypeStruct((NPAD, HID), f32),
  )(agg, degI, degO, W, p)


def _fin_body(a_ref, dI_ref, p_ref, wa_ref, ba_ref, z_ref, aw_ref):
  t = _post(a_ref, dI_ref, p_ref)
  aw = jnp.dot(t, wa_ref[...], preferred_element_type=f32) + ba_ref[...]
  w = jax.nn.sigmoid(aw)
  rows = pl.program_id(0) * R + lax.broadcasted_iota(i32, (R, 1), 0)
  z = jnp.where(rows < N, t * w, 0.0)
  z_ref[...] = z
  aw_ref[...] = aw


def _fin_call(agg, degI, p, Wa, ba):
  return pl.pallas_call(
      _fin_body,
      grid=(NB,),
      in_specs=[
          pl.BlockSpec((R, HID), lambda i: (i, 0)),
          pl.BlockSpec((R, 1), lambda i: (i, 0)),
          pl.BlockSpec((3, HID), lambda i: (0, 0)),
          pl.BlockSpec((HID, 1), lambda i: (0, 0)),
          pl.BlockSpec((1, 1), lambda i: (0, 0)),
      ],
      out_specs=[
          pl.BlockSpec((R, HID), lambda i: (i, 0)),
          pl.BlockSpec((R, 1), lambda i: (i, 0)),
      ],
      out_shape=[
          jax.ShapeDtypeStruct((NPAD, HID), f32),
          jax.ShapeDtypeStruct((NPAD, 1), f32),
      ],
  )(agg, degI, p, Wa, ba)


def _head_body(s_ref, w1_ref, p1_ref, w2_ref, p2_ref, w3_ref, b3_ref, o_ref):
  s = s_ref[0] + s_ref[1]
  y = jnp.dot(s, w1_ref[...], preferred_element_type=f32) + p1_ref[0:1, :]
  y = y * (BN_S * p1_ref[1:2, :]) + p1_ref[2:3, :]
  y = jnp.maximum(y, 0.0)
  y = jnp.dot(y, w2_ref[...], preferred_element_type=f32) + p2_ref[0:1, :]
  y = y * (BN_S * p2_ref[1:2, :]) + p2_ref[2:3, :]
  y = jnp.maximum(y, 0.0)
  y = jnp.dot(y, w3_ref[...], preferred_element_type=f32) + b3_ref[...]
  o_ref[...] = jax.nn.sigmoid(y)


def _head_call(segs, Wfc1, p1, Wl, p2, Wfc2, bfc2):
  return pl.pallas_call(
      _head_body,
      out_shape=jax.ShapeDtypeStruct((G, 67), f32),
  )(segs, Wfc1, p1, Wl, p2, Wfc2, bfc2)


def kernel(h, edge_index, graph_ids, W1, b1, bn1g, bn1b, W2, b2, bn2g, bn2b,
           Wa, ba, Wfc1, bfc1, bnf1g, bnf1b, Wl, bl, bnlg, bnlb, Wfc2, bfc2):
  e3d = edge_index.reshape(2, EROWS, EC)
  dst3d = e3d[1]
  src3d = _shift_call(edge_index).reshape(2, EROWS, EC)

  degs = _deg_call(e3d)
  degO = degs[0].reshape(NPAD, 1)
  degI = degs[1].reshape(NPAD, 1)

  h_pad = jnp.pad(h, ((0, NPAD - N), (0, 0)))

  xs = _l1_call(h_pad, degO, W1)
  agg = _scat_call(xs.reshape(2 * NPAD, 32), src3d, dst3d).reshape(NPAD, HID)
  p = jnp.stack([b1, bn1g, bn1b])
  xs = _mid_call(agg, degI, degO, W2[0], p)
  agg = _scat_call(xs.reshape(2 * NPAD, 32), src3d, dst3d).reshape(NPAD, HID)
  p = jnp.stack([b2[0], bn2g[0], bn2b[0]])
  xs = _mid_call(agg, degI, degO, W2[1], p)
  agg = _scat_call(xs.reshape(2 * NPAD, 32), src3d, dst3d).reshape(NPAD, HID)

  p = jnp.stack([b2[1], bn2g[1], bn2b[1]])
  z, aw = _fin_call(agg, degI, p, Wa, ba.reshape(1, 1))

  gidflat = jnp.pad(graph_ids, (0, NPAD - N))
  segs = _seg_call(z, gidflat)

  p1 = jnp.stack([bfc1, bnf1g, bnf1b])
  p2 = jnp.stack([bl, bnlg, bnlb])
  y = _head_call(segs, Wfc1, p1, Wl, p2, Wfc2, bfc2.reshape(1, 67))
  return (y, aw[:N])


# speedup vs baseline: 1.7856x; 1.7856x over previous
import jax
import jax.numpy as jnp
from jax import lax
from jax.experimental import pallas as pl
from jax.experimental.pallas import tpu as pltpu
from jax.experimental.pallas import tpu_sc as plsc

N = 50000
E = 800000
G = 256
HID = 64
NPAD = 50176
NC = 2
NS = 16
BN_S = 1.0 / (1.0 + 1e-5) ** 0.5

f32 = jnp.float32
i32 = jnp.int32

_SC_MESH = plsc.VectorSubcoreMesh(core_axis_name="c", subcore_axis_name="s")
_SC_PARAMS = pltpu.CompilerParams(use_tc_tiling_on_sc=False)

EC = 80
EROWS = E // EC
T_EDGES = E // NS
TROWS = T_EDGES // EC
GRP = 5
T_GRPS = TROWS // GRP
NPAIR = (T_GRPS - 1) // 2
DGRP = 25

RPT = NPAD // NS
ZB = 56
ZCOP = RPT // ZB


def _zero2d(ref, nrows, ncols):
  def body(r, _):
    for cc in range(ncols // 16):
      ref[r, pl.ds(cc * 16, 16)] = jnp.zeros((16,), f32)
    return 0
  lax.fori_loop(0, nrows, body, 0)


def _deg_body(e3d, out, ones_v, didx, zbuf, dacc, sem):
  c = lax.axis_index("c")
  t = lax.axis_index("s")
  for j in range(EC // 16):
    ones_v[pl.ds(j * 16, 16)] = jnp.ones((16,), f32)
  def zb(i, _):
    zbuf[pl.ds(i * 16, 16)] = jnp.zeros((16,), f32)
    return 0
  lax.fori_loop(0, RPT // 16, zb, 0)
  pltpu.sync_copy(zbuf, dacc.at[pl.ds(t * RPT, RPT)])
  plsc.subcore_barrier()
  def grp(i, _):
    rbase = t * TROWS + i * DGRP
    pltpu.sync_copy(e3d.at[c, pl.ds(rbase, DGRP)], didx)
    descs = [
        pltpu.async_copy(ones_v, dacc.at[didx.at[j]], sem, add=True)
        for j in range(DGRP)
    ]
    for d in descs:
      d.wait()
    return 0
  lax.fori_loop(0, TROWS // DGRP, grp, 0)
  plsc.subcore_barrier()
  pltpu.sync_copy(dacc.at[pl.ds(t * RPT, RPT)], out.at[c, pl.ds(t * RPT, RPT)])


_deg_call = pl.kernel(
    _deg_body,
    out_type=jax.ShapeDtypeStruct((NC, NPAD), f32),
    mesh=_SC_MESH,
    compiler_params=_SC_PARAMS,
    scratch_types=[
        pltpu.VMEM((EC,), f32),
        pltpu.VMEM((DGRP, EC), i32),
        pltpu.VMEM((RPT,), f32),
        pltpu.VMEM_SHARED((NPAD,), f32),
        pltpu.SemaphoreType.DMA,
    ],
)


def _scat_body(xs, src3d, dst3d, out,
               sidxA, didxA, rowsA, sidxB, didxB, rowsB, zbuf, acc,
               gsemA, ssemA, isemA, gsemB, ssemB, isemB):
  c = lax.axis_index("c")
  t = lax.axis_index("s")
  _zero2d(zbuf, ZB, 32)
  zd = [pltpu.async_copy(zbuf, acc.at[pl.ds((t * ZCOP + k) * ZB, ZB)], ssemA)
        for k in range(ZCOP)]
  for d in zd:
    d.wait()
  plsc.subcore_barrier()

  def fire_idx(g, sidx, didx, isem):
    rbase = t * TROWS + g * GRP
    return (pltpu.async_copy(src3d.at[c, pl.ds(rbase, GRP)], sidx, isem),
            pltpu.async_copy(dst3d.at[pl.ds(rbase, GRP)], didx, isem))

  def fire_gathers(sidx, rows, gsem):
    for j in range(GRP):
      pltpu.async_copy(xs.at[sidx.at[j]], rows.at[pl.ds(j * EC, EC)], gsem)

  def drain_gathers(sidx, rows, gsem):
    for j in range(GRP):
      pltpu.make_async_copy(xs.at[sidx.at[j]], rows.at[pl.ds(j * EC, EC)],
                            gsem).wait()

  def fire_scats(rows, didx, ssem):
    for j in range(GRP):
      pltpu.async_copy(rows.at[pl.ds(j * EC, EC)], acc.at[didx.at[j]], ssem,
                       add=True)

  def drain_scats(rows, ssem):
    for j in range(GRP):
      pltpu.make_async_copy(xs.at[pl.ds(0, EC)], rows.at[pl.ds(j * EC, EC)],
                            ssem).wait()

  i0, i1 = fire_idx(0, sidxA, didxA, isemA)
  i0.wait()
  i1.wait()
  fire_gathers(sidxA, rowsA, gsemA)

  def pair(i, _):
    gB = 2 * i + 1
    ib0, ib1 = fire_idx(gB, sidxB, didxB, isemB)
    drain_gathers(sidxA, rowsA, gsemA)
    fire_scats(rowsA, didxA, ssemA)
    ib0.wait()
    ib1.wait()
    fire_gathers(sidxB, rowsB, gsemB)
    drain_scats(rowsA, ssemA)
    ia0, ia1 = fire_idx(gB + 1, sidxA, didxA, isemA)
    drain_gathers(sidxB, rowsB, gsemB)
    fire_scats(rowsB, didxB, ssemB)
    ia0.wait()
    ia1.wait()
    fire_gathers(sidxA, rowsA, gsemA)
    drain_scats(rowsB, ssemB)
    return 0
  lax.fori_loop(0, NPAIR, pair, 0)

  drain_gathers(sidxA, rowsA, gsemA)
  fire_scats(rowsA, didxA, ssemA)
  drain_scats(rowsA, ssemA)

  plsc.subcore_barrier()
  wb = [pltpu.async_copy(acc.at[pl.ds((t * ZCOP + k) * ZB, ZB)],
                         out.at[pl.ds((t * ZCOP + k) * ZB, ZB), c], gsemA)
        for k in range(ZCOP)]
  for d in wb:
    d.wait()


_scat_call = pl.kernel(
    _scat_body,
    out_type=jax.ShapeDtypeStruct((NPAD, NC, 32), f32),
    mesh=_SC_MESH,
    compiler_params=_SC_PARAMS,
    scratch_types=[
        pltpu.VMEM((GRP, EC), i32),
        pltpu.VMEM((GRP, EC), i32),
        pltpu.VMEM((GRP * EC, 32), f32),
        pltpu.VMEM((GRP, EC), i32),
        pltpu.VMEM((GRP, EC), i32),
        pltpu.VMEM((GRP * EC, 32), f32),
        pltpu.VMEM((ZB, 32), f32),
        pltpu.VMEM_SHARED((NPAD, 32), f32),
        pltpu.SemaphoreType.DMA,
        pltpu.SemaphoreType.DMA,
        pltpu.SemaphoreType.DMA,
        pltpu.SemaphoreType.DMA,
        pltpu.SemaphoreType.DMA,
        pltpu.SemaphoreType.DMA,
    ],
)


SEGC = 112
SEG_TROWS = NPAD // (NC * NS) // SEGC

def _seg_body(z, gidflat, out, didx, rows, zbuf, acc):
  c = lax.axis_index("c")
  t = lax.axis_index("s")
  w = c * NS + t
  _zero2d(zbuf, 16, HID)
  pltpu.sync_copy(zbuf, acc.at[pl.ds(t * 16, 16)])
  plsc.subcore_barrier()
  def body(i, _):
    start = (w * SEG_TROWS + i) * SEGC
    pltpu.sync_copy(gidflat.at[pl.ds(start, SEGC)], didx)
    pltpu.sync_copy(z.at[pl.ds(start, SEGC)], rows)
    pltpu.sync_copy(rows, acc.at[didx], add=True)
    return 0
  lax.fori_loop(0, SEG_TROWS, body, 0)
  plsc.subcore_barrier()
  pltpu.sync_copy(acc.at[pl.ds(t * 16, 16)], out.at[c, pl.ds(t * 16, 16)])


_seg_call = pl.kernel(
    _seg_body,
    out_type=jax.ShapeDtypeStruct((NC, G, HID), f32),
    mesh=_SC_MESH,
    compiler_params=_SC_PARAMS,
    scratch_types=[
        pltpu.VMEM((SEGC,), i32),
        pltpu.VMEM((SEGC, HID), f32),
        pltpu.VMEM((16, HID), f32),
        pltpu.VMEM_SHARED((G, HID), f32),
    ],
)


EB = 80000

def _shift_body(e_ref, o_ref):
  s = e_ref[0]
  o_ref[0:1] = 2 * s
  o_ref[1:2] = 2 * s + 1


def _shift_call(edge_index):
  return pl.pallas_call(
      _shift_body,
      grid=(E // EB,),
      in_specs=[pl.BlockSpec((1, 1, EB), lambda i: (0, 0, i))],
      out_specs=pl.BlockSpec((2, EB), lambda i: (0, i)),
      out_shape=jax.ShapeDtypeStruct((2, E), i32),
  )(edge_index.reshape(2, 1, E))


NH = NPAD // 2
R = 512
NB = NH // R


def _expand(v2):
  lane = lax.broadcasted_iota(i32, (R, 128), 1)
  return jnp.where(lane < 64, v2[:, 0:1], v2[:, 1:2])


def _l1_body(h_ref, d_ref, w_ref, o_ref):
  dois = _expand(lax.rsqrt(jnp.maximum(d_ref[...], 1.0)))
  o_ref[...] = jnp.dot(h_ref[...], w_ref[...], preferred_element_type=f32) * dois


def _l1_call(h2, degO2, W1x):
  return pl.pallas_call(
      _l1_body,
      grid=(NB,),
      in_specs=[
          pl.BlockSpec((R, 76), lambda i: (i, 0)),
          pl.BlockSpec((R, 2), lambda i: (i, 0)),
          pl.BlockSpec((76, 128), lambda i: (0, 0)),
      ],
      out_specs=pl.BlockSpec((R, 128), lambda i: (i, 0)),
      out_shape=jax.ShapeDtypeStruct((NH, 128), f32),
  )(h2, degO2, W1x)


def _post(a_ref, dI_ref, p_ref):
  diis = _expand(lax.rsqrt(jnp.maximum(dI_ref[...], 1.0)))
  t = a_ref[...] * diis + p_ref[0:1, :]
  t = t * (BN_S * p_ref[1:2, :]) + p_ref[2:3, :]
  return jnp.maximum(t, 0.0)


def _mid_body(a_ref, dI_ref, dO_ref, w_ref, p_ref, o_ref):
  t = _post(a_ref, dI_ref, p_ref)
  dois = _expand(lax.rsqrt(jnp.maximum(dO_ref[...], 1.0)))
  o_ref[...] = jnp.dot(t, w_ref[...], preferred_element_type=f32) * dois


def _mid_call(agg2, degI2, degO2, Wx, p128):
  return pl.pallas_call(
      _mid_body,
      grid=(NB,),
      in_specs=[
          pl.BlockSpec((R, 128), lambda i: (i, 0)),
          pl.BlockSpec((R, 2), lambda i: (i, 0)),
          pl.BlockSpec((R, 2), lambda i: (i, 0)),
          pl.BlockSpec((128, 128), lambda i: (0, 0)),
          pl.BlockSpec((3, 128), lambda i: (0, 0)),
      ],
      out_specs=pl.BlockSpec((R, 128), lambda i: (i, 0)),
      out_shape=jax.ShapeDtypeStruct((NH, 128), f32),
  )(agg2, degI2, degO2, Wx, p128)


def _fin_body(a_ref, dI_ref, p_ref, wa_ref, ba_ref, z_ref, aw_ref):
  t = _post(a_ref, dI_ref, p_ref)
  aw2 = jnp.dot(t, wa_ref[...], preferred_element_type=f32) + ba_ref[...]
  w128 = _expand(jax.nn.sigmoid(aw2))
  lane = lax.broadcasted_iota(i32, (R, 128), 1)
  rows = pl.program_id(0) * R + lax.broadcasted_iota(i32, (R, 128), 0)
  nid = 2 * rows + jnp.where(lane < 64, 0, 1)
  z_ref[...] = jnp.where(nid < N, t * w128, 0.0)
  aw_ref[...] = aw2


def _fin_call(agg2, degI2, p128, Wax, ba):
  return pl.pallas_call(
      _fin_body,
      grid=(NB,),
      in_specs=[
          pl.BlockSpec((R, 128), lambda i: (i, 0)),
          pl.BlockSpec((R, 2), lambda i: (i, 0)),
          pl.BlockSpec((3, 128), lambda i: (0, 0)),
          pl.BlockSpec((128, 2), lambda i: (0, 0)),
          pl.BlockSpec((1, 1), lambda i: (0, 0)),
      ],
      out_specs=[
          pl.BlockSpec((R, 128), lambda i: (i, 0)),
          pl.BlockSpec((R, 2), lambda i: (i, 0)),
      ],
      out_shape=[
          jax.ShapeDtypeStruct((NH, 128), f32),
          jax.ShapeDtypeStruct((NH, 2), f32),
      ],
  )(agg2, degI2, p128, Wax, ba)


def _head_body(s_ref, w1_ref, p1_ref, w2_ref, p2_ref, w3_ref, b3_ref, o_ref):
  s = s_ref[0] + s_ref[1]
  y = jnp.dot(s, w1_ref[...], preferred_element_type=f32) + p1_ref[0:1, :]
  y = y * (BN_S * p1_ref[1:2, :]) + p1_ref[2:3, :]
  y = jnp.maximum(y, 0.0)
  y = jnp.dot(y, w2_ref[...], preferred_element_type=f32) + p2_ref[0:1, :]
  y = y * (BN_S * p2_ref[1:2, :]) + p2_ref[2:3, :]
  y = jnp.maximum(y, 0.0)
  y = jnp.dot(y, w3_ref[...], preferred_element_type=f32) + b3_ref[...]
  o_ref[...] = jax.nn.sigmoid(y)


def _head_call(segs, Wfc1, p1, Wl, p2, Wfc2, bfc2):
  return pl.pallas_call(
      _head_body,
      out_shape=jax.ShapeDtypeStruct((G, 67), f32),
  )(segs, Wfc1, p1, Wl, p2, Wfc2, bfc2)


def kernel(h, edge_index, graph_ids, W1, b1, bn1g, bn1b, W2, b2, bn2g, bn2b,
           Wa, ba, Wfc1, bfc1, bnf1g, bnf1b, Wl, bl, bnlg, bnlb, Wfc2, bfc2):
  e3d = edge_index.reshape(2, EROWS, EC)
  dst3d = e3d[1]
  src3d = _shift_call(edge_index).reshape(2, EROWS, EC)

  degs = _deg_call(e3d)
  degO2 = degs[0].reshape(NH, 2)
  degI2 = degs[1].reshape(NH, 2)

  h2 = jnp.pad(h, ((0, NPAD - N), (0, 0))).reshape(NH, 76)
  W1x = jnp.zeros((76, 128), f32).at[:38, :64].set(W1).at[38:, 64:].set(W1)
  W2x = [jnp.zeros((128, 128), f32).at[:64, :64].set(W2[i]).at[64:, 64:].set(W2[i])
         for i in range(2)]
  Wax = jnp.zeros((128, 2), f32).at[:64, 0].set(Wa[:, 0]).at[64:, 1].set(Wa[:, 0])

  xs = _l1_call(h2, degO2, W1x)
  agg2 = _scat_call(xs.reshape(2 * NPAD, 32), src3d, dst3d).reshape(NH, 128)
  p = jnp.tile(jnp.stack([b1, bn1g, bn1b]), (1, 2))
  xs = _mid_call(agg2, degI2, degO2, W2x[0], p)
  agg2 = _scat_call(xs.reshape(2 * NPAD, 32), src3d, dst3d).reshape(NH, 128)
  p = jnp.tile(jnp.stack([b2[0], bn2g[0], bn2b[0]]), (1, 2))
  xs = _mid_call(agg2, degI2, degO2, W2x[1], p)
  agg2 = _scat_call(xs.reshape(2 * NPAD, 32), src3d, dst3d).reshape(NH, 128)

  p = jnp.tile(jnp.stack([b2[1], bn2g[1], bn2b[1]]), (1, 2))
  z2, aw2 = _fin_call(agg2, degI2, p, Wax, ba.reshape(1, 1))
  aw = aw2.reshape(NPAD, 1)

  gidflat = jnp.pad(graph_ids, (0, NPAD - N))
  segs = _seg_call(z2.reshape(NPAD, HID), gidflat)

  p1 = jnp.stack([bfc1, bnf1g, bnf1b])
  p2 = jnp.stack([bl, bnlg, bnlb])
  y = _head_call(segs, Wfc1, p1, Wl, p2, Wfc2, bfc2.reshape(1, 67))
  return (y, aw[:N])


# speedup vs baseline: 1.8923x; 1.0598x over previous
import jax
import jax.numpy as jnp
from jax import lax
from jax.experimental import pallas as pl
from jax.experimental.pallas import tpu as pltpu
from jax.experimental.pallas import tpu_sc as plsc

N = 50000
E = 800000
G = 256
HID = 64
NPAD = 50176
NC = 2
NS = 16
BN_S = 1.0 / (1.0 + 1e-5) ** 0.5

f32 = jnp.float32
i32 = jnp.int32

_SC_MESH = plsc.VectorSubcoreMesh(core_axis_name="c", subcore_axis_name="s")
_SC_PARAMS = pltpu.CompilerParams(use_tc_tiling_on_sc=False)

EC = 80
EROWS = E // EC
T_EDGES = E // NS
TROWS = T_EDGES // EC
GRP = 5
T_GRPS = TROWS // GRP
NPAIR = (T_GRPS - 1) // 2
DGRP = 125

RPT = NPAD // NS
ZB = 56
ZCOP = RPT // ZB


def _zero2d(ref, nrows, ncols):
  def body(r, _):
    for cc in range(ncols // 16):
      ref[r, pl.ds(cc * 16, 16)] = jnp.zeros((16,), f32)
    return 0
  lax.fori_loop(0, nrows, body, 0)


def _deg_body(e3d, outO, outI, ones_v, didx, zbuf, dacc, sem):
  c = lax.axis_index("c")
  t = lax.axis_index("s")
  for j in range(EC // 16):
    ones_v[pl.ds(j * 16, 16)] = jnp.ones((16,), f32)
  def zb(i, _):
    zbuf[pl.ds(i * 16, 16)] = jnp.zeros((16,), f32)
    return 0
  lax.fori_loop(0, RPT // 16, zb, 0)
  pltpu.sync_copy(zbuf, dacc.at[pl.ds(t * RPT, RPT)])
  plsc.subcore_barrier()
  def grp(i, _):
    rbase = t * TROWS + i * DGRP
    pltpu.sync_copy(e3d.at[c, pl.ds(rbase, DGRP)], didx)
    descs = [
        pltpu.async_copy(ones_v, dacc.at[didx.at[j]], sem, add=True)
        for j in range(DGRP)
    ]
    for d in descs:
      d.wait()
    return 0
  lax.fori_loop(0, TROWS // DGRP, grp, 0)
  plsc.subcore_barrier()

  @pl.when(c == 0)
  def _():
    pltpu.sync_copy(dacc.at[pl.ds(t * RPT, RPT)], outO.at[pl.ds(t * RPT, RPT)])

  @pl.when(c == 1)
  def _():
    pltpu.sync_copy(dacc.at[pl.ds(t * RPT, RPT)], outI.at[pl.ds(t * RPT, RPT)])


_deg_call = pl.kernel(
    _deg_body,
    out_type=[jax.ShapeDtypeStruct((NPAD,), f32),
              jax.ShapeDtypeStruct((NPAD,), f32)],
    mesh=_SC_MESH,
    compiler_params=_SC_PARAMS,
    scratch_types=[
        pltpu.VMEM((EC,), f32),
        pltpu.VMEM((DGRP, EC), i32),
        pltpu.VMEM((RPT,), f32),
        pltpu.VMEM_SHARED((NPAD,), f32),
        pltpu.SemaphoreType.DMA,
    ],
)


def _scat_body(xs, src3d, dst3d, out,
               sidxA, didxA, rowsA, sidxB, didxB, rowsB, zbuf, acc,
               gsemA, ssemA, isemA, gsemB, ssemB, isemB):
  c = lax.axis_index("c")
  t = lax.axis_index("s")
  _zero2d(zbuf, ZB, 32)
  zd = [pltpu.async_copy(zbuf, acc.at[pl.ds((t * ZCOP + k) * ZB, ZB)], ssemA)
        for k in range(ZCOP)]
  for d in zd:
    d.wait()
  plsc.subcore_barrier()

  def fire_idx(g, sidx, didx, isem):
    rbase = t * TROWS + g * GRP
    return (pltpu.async_copy(src3d.at[c, pl.ds(rbase, GRP)], sidx, isem),
            pltpu.async_copy(dst3d.at[pl.ds(rbase, GRP)], didx, isem))

  def fire_gathers(sidx, rows, gsem):
    for j in range(GRP):
      pltpu.async_copy(xs.at[sidx.at[j]], rows.at[pl.ds(j * EC, EC)], gsem)

  def drain_gathers(sidx, rows, gsem):
    for j in range(GRP):
      pltpu.make_async_copy(xs.at[sidx.at[j]], rows.at[pl.ds(j * EC, EC)],
                            gsem).wait()

  def fire_scats(rows, didx, ssem):
    for j in range(GRP):
      pltpu.async_copy(rows.at[pl.ds(j * EC, EC)], acc.at[didx.at[j]], ssem,
                       add=True)

  def drain_scats(rows, ssem):
    for j in range(GRP):
      pltpu.make_async_copy(xs.at[pl.ds(0, EC)], rows.at[pl.ds(j * EC, EC)],
                            ssem).wait()

  i0, i1 = fire_idx(0, sidxA, didxA, isemA)
  i0.wait()
  i1.wait()
  fire_gathers(sidxA, rowsA, gsemA)

  def pair(i, _):
    gB = 2 * i + 1
    ib0, ib1 = fire_idx(gB, sidxB, didxB, isemB)
    drain_gathers(sidxA, rowsA, gsemA)
    fire_scats(rowsA, didxA, ssemA)
    ib0.wait()
    ib1.wait()
    fire_gathers(sidxB, rowsB, gsemB)
    drain_scats(rowsA, ssemA)
    ia0, ia1 = fire_idx(gB + 1, sidxA, didxA, isemA)
    drain_gathers(sidxB, rowsB, gsemB)
    fire_scats(rowsB, didxB, ssemB)
    ia0.wait()
    ia1.wait()
    fire_gathers(sidxA, rowsA, gsemA)
    drain_scats(rowsB, ssemB)
    return 0
  lax.fori_loop(0, NPAIR, pair, 0)

  drain_gathers(sidxA, rowsA, gsemA)
  fire_scats(rowsA, didxA, ssemA)
  drain_scats(rowsA, ssemA)

  plsc.subcore_barrier()
  wb = [pltpu.async_copy(acc.at[pl.ds((t * ZCOP + k) * ZB, ZB)],
                         out.at[pl.ds((t * ZCOP + k) * ZB, ZB), c], gsemA)
        for k in range(ZCOP)]
  for d in wb:
    d.wait()


_scat_call = pl.kernel(
    _scat_body,
    out_type=jax.ShapeDtypeStruct((NPAD, NC, 32), f32),
    mesh=_SC_MESH,
    compiler_params=_SC_PARAMS,
    scratch_types=[
        pltpu.VMEM((GRP, EC), i32),
        pltpu.VMEM((GRP, EC), i32),
        pltpu.VMEM((GRP * EC, 32), f32),
        pltpu.VMEM((GRP, EC), i32),
        pltpu.VMEM((GRP, EC), i32),
        pltpu.VMEM((GRP * EC, 32), f32),
        pltpu.VMEM((ZB, 32), f32),
        pltpu.VMEM_SHARED((NPAD, 32), f32),
        pltpu.SemaphoreType.DMA,
        pltpu.SemaphoreType.DMA,
        pltpu.SemaphoreType.DMA,
        pltpu.SemaphoreType.DMA,
        pltpu.SemaphoreType.DMA,
        pltpu.SemaphoreType.DMA,
    ],
)


SEGC = 112
SEG_TROWS = NPAD // (NC * NS) // SEGC

def _seg_body(z, gidflat, out, didx, rows, zbuf, acc):
  c = lax.axis_index("c")
  t = lax.axis_index("s")
  w = c * NS + t
  _zero2d(zbuf, 16, HID)
  pltpu.sync_copy(zbuf, acc.at[pl.ds(t * 16, 16)])
  plsc.subcore_barrier()
  def body(i, _):
    start = (w * SEG_TROWS + i) * SEGC
    pltpu.sync_copy(gidflat.at[pl.ds(start, SEGC)], didx)
    pltpu.sync_copy(z.at[pl.ds(start, SEGC)], rows)
    pltpu.sync_copy(rows, acc.at[didx], add=True)
    return 0
  lax.fori_loop(0, SEG_TROWS, body, 0)
  plsc.subcore_barrier()
  pltpu.sync_copy(acc.at[pl.ds(t * 16, 16)], out.at[c, pl.ds(t * 16, 16)])


_seg_call = pl.kernel(
    _seg_body,
    out_type=jax.ShapeDtypeStruct((NC, G, HID), f32),
    mesh=_SC_MESH,
    compiler_params=_SC_PARAMS,
    scratch_types=[
        pltpu.VMEM((SEGC,), i32),
        pltpu.VMEM((SEGC, HID), f32),
        pltpu.VMEM((16, HID), f32),
        pltpu.VMEM_SHARED((G, HID), f32),
    ],
)


EB = 80000

def _shift_body(e_ref, o_ref):
  s = e_ref[0]
  o_ref[0:1] = 2 * s
  o_ref[1:2] = 2 * s + 1


def _shift_call(edge_index):
  return pl.pallas_call(
      _shift_body,
      grid=(E // EB,),
      in_specs=[pl.BlockSpec((1, 1, EB), lambda i: (0, 0, i))],
      out_specs=pl.BlockSpec((2, EB), lambda i: (0, i)),
      out_shape=jax.ShapeDtypeStruct((2, E), i32),
  )(edge_index.reshape(2, 1, E))


NH = NPAD // 2
R = 896
NB = NH // R


def _expand(v2):
  lane = lax.broadcasted_iota(i32, (R, 128), 1)
  return jnp.where(lane < 64, v2[:, 0:1], v2[:, 1:2])


def _l1_body(h_ref, d_ref, w_ref, o_ref):
  dois = _expand(lax.rsqrt(jnp.maximum(d_ref[...], 1.0)))
  o_ref[...] = jnp.dot(h_ref[...], w_ref[...], preferred_element_type=f32) * dois


def _l1_call(h2, degO2, W1x):
  return pl.pallas_call(
      _l1_body,
      grid=(NB,),
      in_specs=[
          pl.BlockSpec((R, 76), lambda i: (i, 0)),
          pl.BlockSpec((R, 2), lambda i: (i, 0)),
          pl.BlockSpec((76, 128), lambda i: (0, 0)),
      ],
      out_specs=pl.BlockSpec((R, 128), lambda i: (i, 0)),
      out_shape=jax.ShapeDtypeStruct((NH, 128), f32),
  )(h2, degO2, W1x)


def _post(a_ref, dI_ref, p_ref):
  diis = _expand(lax.rsqrt(jnp.maximum(dI_ref[...], 1.0)))
  t = a_ref[...] * diis + p_ref[0:1, :]
  t = t * (BN_S * p_ref[1:2, :]) + p_ref[2:3, :]
  return jnp.maximum(t, 0.0)


def _mid_body(a_ref, dI_ref, dO_ref, w_ref, p_ref, o_ref):
  t = _post(a_ref, dI_ref, p_ref)
  dois = _expand(lax.rsqrt(jnp.maximum(dO_ref[...], 1.0)))
  o_ref[...] = jnp.dot(t, w_ref[...], preferred_element_type=f32) * dois


def _mid_call(agg2, degI2, degO2, Wx, p128):
  return pl.pallas_call(
      _mid_body,
      grid=(NB,),
      in_specs=[
          pl.BlockSpec((R, 128), lambda i: (i, 0)),
          pl.BlockSpec((R, 2), lambda i: (i, 0)),
          pl.BlockSpec((R, 2), lambda i: (i, 0)),
          pl.BlockSpec((128, 128), lambda i: (0, 0)),
          pl.BlockSpec((3, 128), lambda i: (0, 0)),
      ],
      out_specs=pl.BlockSpec((R, 128), lambda i: (i, 0)),
      out_shape=jax.ShapeDtypeStruct((NH, 128), f32),
  )(agg2, degI2, degO2, Wx, p128)


def _fin_body(a_ref, dI_ref, p_ref, wa_ref, ba_ref, z_ref, aw_ref):
  t = _post(a_ref, dI_ref, p_ref)
  aw2 = jnp.dot(t, wa_ref[...], preferred_element_type=f32) + ba_ref[...]
  w128 = _expand(jax.nn.sigmoid(aw2))
  lane = lax.broadcasted_iota(i32, (R, 128), 1)
  rows = pl.program_id(0) * R + lax.broadcasted_iota(i32, (R, 128), 0)
  nid = 2 * rows + jnp.where(lane < 64, 0, 1)
  z_ref[...] = jnp.where(nid < N, t * w128, 0.0)
  aw_ref[...] = aw2


def _fin_call(agg2, degI2, p128, Wax, ba):
  return pl.pallas_call(
      _fin_body,
      grid=(NB,),
      in_specs=[
          pl.BlockSpec((R, 128), lambda i: (i, 0)),
          pl.BlockSpec((R, 2), lambda i: (i, 0)),
          pl.BlockSpec((3, 128), lambda i: (0, 0)),
          pl.BlockSpec((128, 2), lambda i: (0, 0)),
          pl.BlockSpec((1, 1), lambda i: (0, 0)),
      ],
      out_specs=[
          pl.BlockSpec((R, 128), lambda i: (i, 0)),
          pl.BlockSpec((R, 2), lambda i: (i, 0)),
      ],
      out_shape=[
          jax.ShapeDtypeStruct((NH, 128), f32),
          jax.ShapeDtypeStruct((NH, 2), f32),
      ],
  )(agg2, degI2, p128, Wax, ba)


def _head_body(s_ref, w1_ref, p1_ref, w2_ref, p2_ref, w3_ref, b3_ref, o_ref):
  s = s_ref[0] + s_ref[1]
  y = jnp.dot(s, w1_ref[...], preferred_element_type=f32) + p1_ref[0:1, :]
  y = y * (BN_S * p1_ref[1:2, :]) + p1_ref[2:3, :]
  y = jnp.maximum(y, 0.0)
  y = jnp.dot(y, w2_ref[...], preferred_element_type=f32) + p2_ref[0:1, :]
  y = y * (BN_S * p2_ref[1:2, :]) + p2_ref[2:3, :]
  y = jnp.maximum(y, 0.0)
  y = jnp.dot(y, w3_ref[...], preferred_element_type=f32) + b3_ref[...]
  o_ref[...] = jax.nn.sigmoid(y)


def _head_call(segs, Wfc1, p1, Wl, p2, Wfc2, bfc2):
  return pl.pallas_call(
      _head_body,
      out_shape=jax.ShapeDtypeStruct((G, 67), f32),
  )(segs, Wfc1, p1, Wl, p2, Wfc2, bfc2)


def kernel(h, edge_index, graph_ids, W1, b1, bn1g, bn1b, W2, b2, bn2g, bn2b,
           Wa, ba, Wfc1, bfc1, bnf1g, bnf1b, Wl, bl, bnlg, bnlb, Wfc2, bfc2):
  e3d = edge_index.reshape(2, EROWS, EC)
  dst3d = e3d[1]
  src3d = _shift_call(edge_index).reshape(2, EROWS, EC)

  degO, degI = _deg_call(e3d)
  degO2 = degO.reshape(NH, 2)
  degI2 = degI.reshape(NH, 2)

  h2 = jnp.pad(h, ((0, NPAD - N), (0, 0))).reshape(NH, 76)
  W1x = jnp.zeros((76, 128), f32).at[:38, :64].set(W1).at[38:, 64:].set(W1)
  W2x = [jnp.zeros((128, 128), f32).at[:64, :64].set(W2[i]).at[64:, 64:].set(W2[i])
         for i in range(2)]
  Wax = jnp.zeros((128, 2), f32).at[:64, 0].set(Wa[:, 0]).at[64:, 1].set(Wa[:, 0])

  xs = _l1_call(h2, degO2, W1x)
  agg2 = _scat_call(xs.reshape(2 * NPAD, 32), src3d, dst3d).reshape(NH, 128)
  p = jnp.tile(jnp.stack([b1, bn1g, bn1b]), (1, 2))
  xs = _mid_call(agg2, degI2, degO2, W2x[0], p)
  agg2 = _scat_call(xs.reshape(2 * NPAD, 32), src3d, dst3d).reshape(NH, 128)
  p = jnp.tile(jnp.stack([b2[0], bn2g[0], bn2b[0]]), (1, 2))
  xs = _mid_call(agg2, degI2, degO2, W2x[1], p)
  agg2 = _scat_call(xs.reshape(2 * NPAD, 32), src3d, dst3d).reshape(NH, 128)

  p = jnp.tile(jnp.stack([b2[1], bn2g[1], bn2b[1]]), (1, 2))
  z2, aw2 = _fin_call(agg2, degI2, p, Wax, ba.reshape(1, 1))
  aw = aw2.reshape(NPAD, 1)

  gidflat = jnp.pad(graph_ids, (0, NPAD - N))
  segs = _seg_call(z2.reshape(NPAD, HID), gidflat)

  p1 = jnp.stack([bfc1, bnf1g, bnf1b])
  p2 = jnp.stack([bl, bnlg, bnlb])
  y = _head_call(segs, Wfc1, p1, Wl, p2, Wfc2, bfc2.reshape(1, 67))
  return (y, aw[:N])
